# baseline (speedup 1.0000x reference)
import jax
import jax.numpy as jnp
from jax.experimental import pallas as pl
from jax.experimental.pallas import tpu as pltpu

_B = 16
_R = 576
_K = 64
_C = 1024
_G = 128
_NG = _C // _G
_B_BLK = 8
_MH = _B_BLK * _R // 2
_BIG = 3.0e38


def _argmin_rows(dot2, nx, nc):
    runmin = jnp.full((_MH, _G), _BIG, jnp.float32)
    rung = jnp.zeros((_MH, _G), jnp.int32)
    for g in range(_NG):
        d = (nx + nc[:, g * _G:(g + 1) * _G]) - dot2[:, g * _G:(g + 1) * _G]
        mask = d < runmin
        rung = jnp.where(mask, jnp.int32(g), rung)
        runmin = jnp.minimum(runmin, d)
    m = jnp.min(runmin, axis=1, keepdims=True)
    lane = jax.lax.broadcasted_iota(jnp.int32, (_MH, _G), 1)
    cand = jnp.where(runmin == m, rung * _G + lane, jnp.int32(_C))
    return jnp.min(cand.reshape(_B_BLK, _R // 2, _G), axis=2)


def _assign_kernel(x_ref, ch_ref, cl_ref, out_ref):
    xp = x_ref[...].reshape(_MH, 2 * _K)
    ch = ch_ref[...]
    cl = cl_ref[...]
    xp2 = xp * xp
    nx_e = jnp.sum(xp2[:, :_K], axis=1, keepdims=True)
    nx_o = jnp.sum(xp2[:, _K:], axis=1, keepdims=True)
    nc = jnp.sum(ch * ch, axis=1)[None, :]
    x2 = xp * 2.0
    dot2e = jax.lax.dot_general(
        x2, ch, (((1,), (1,)), ((), ())),
        preferred_element_type=jnp.float32)
    dot2o = jax.lax.dot_general(
        x2, cl, (((1,), (1,)), ((), ())),
        preferred_element_type=jnp.float32)
    idx_e = _argmin_rows(dot2e, nx_e, nc)
    idx_o = _argmin_rows(dot2o, nx_o, nc)
    out = jnp.stack([idx_e, idx_o], axis=-1)
    out_ref[...] = out.reshape(_B_BLK, _R)


def kernel(x, centroids):
    x128 = x.reshape(_B, _R // 2, 2 * _K)
    z = jnp.zeros_like(centroids)
    ch = jnp.concatenate([centroids, z], axis=1)
    cl = jnp.concatenate([z, centroids], axis=1)
    out = pl.pallas_call(
        _assign_kernel,
        grid=(_B // _B_BLK,),
        in_specs=[
            pl.BlockSpec((_B_BLK, _R // 2, 2 * _K), lambda i: (i, 0, 0)),
            pl.BlockSpec((_C, 2 * _K), lambda i: (0, 0)),
            pl.BlockSpec((_C, 2 * _K), lambda i: (0, 0)),
        ],
        out_specs=pl.BlockSpec((_B_BLK, _R), lambda i: (i, 0)),
        out_shape=jax.ShapeDtypeStruct((_B, _R), jnp.int32),
    )(x128, ch, cl)
    return out


# speedup vs baseline: 1.3171x; 1.3171x over previous
import jax
import jax.numpy as jnp
from jax.experimental import pallas as pl
from jax.experimental.pallas import tpu as pltpu

_B = 16
_R = 576
_K = 64
_C = 1024


def _assign_kernel(x_ref, c_ref, out_ref):
    xb = x_ref[0]
    ct = c_ref[...]
    nx = jnp.sum(xb * xb, axis=0)[None, :]
    nc = jnp.sum(ct * ct, axis=0, keepdims=True)
    nc_col = jnp.swapaxes(nc, 0, 1)
    dot2 = jax.lax.dot_general(
        ct + ct, xb, (((0,), (0,)), ((), ())),
        preferred_element_type=jnp.float32,
    )
    d = (nx + nc_col) - dot2
    idx = jnp.argmin(d, axis=0)
    out_ref[...] = idx.reshape(1, 1, _R)


def kernel(x, centroids):
    xt = jnp.swapaxes(x, 1, 2)
    ct = centroids.T
    out = pl.pallas_call(
        _assign_kernel,
        grid=(_B,),
        in_specs=[
            pl.BlockSpec((1, _K, _R), lambda i: (i, 0, 0)),
            pl.BlockSpec((_K, _C), lambda i: (0, 0)),
        ],
        out_specs=pl.BlockSpec((1, 1, _R), lambda i: (i, 0, 0)),
        out_shape=jax.ShapeDtypeStruct((_B, 1, _R), jnp.int32),
    )(xt, ct)
    return out.reshape(_B, _R)


# speedup vs baseline: 1.8226x; 1.3838x over previous
import jax
import jax.numpy as jnp
from jax.experimental import pallas as pl
from jax.experimental.pallas import tpu as pltpu

_B = 16
_R = 576
_K = 64
_C = 1024


_B_BLK = 8


def _assign_kernel(x_ref, c_ref, out_ref):
    ct = c_ref[...]
    nc = jnp.sum(ct * ct, axis=0, keepdims=True)
    nc_col = jnp.swapaxes(nc, 0, 1)
    ct2 = ct + ct
    for b in range(_B_BLK):
        xb = x_ref[b]
        nx = jnp.sum(xb * xb, axis=0)[None, :]
        dot2 = jax.lax.dot_general(
            ct2, xb, (((0,), (0,)), ((), ())),
            preferred_element_type=jnp.float32,
        )
        d = (nx + nc_col) - dot2
        idx = jnp.argmin(d, axis=0)
        out_ref[b, 0, :] = idx


def kernel(x, centroids):
    xt = jnp.swapaxes(x, 1, 2)
    ct = centroids.T
    out = pl.pallas_call(
        _assign_kernel,
        grid=(_B // _B_BLK,),
        in_specs=[
            pl.BlockSpec((_B_BLK, _K, _R), lambda i: (i, 0, 0)),
            pl.BlockSpec((_K, _C), lambda i: (0, 0)),
        ],
        out_specs=pl.BlockSpec((_B_BLK, 1, _R), lambda i: (i, 0, 0)),
        out_shape=jax.ShapeDtypeStruct((_B, 1, _R), jnp.int32),
    )(xt, ct)
    return out.reshape(_B, _R)


# speedup vs baseline: 1.8445x; 1.0120x over previous
import jax
import jax.numpy as jnp
from jax.experimental import pallas as pl
from jax.experimental.pallas import tpu as pltpu

_B = 16
_R = 576
_K = 64
_C = 1024


_B_BLK = 16


def _assign_kernel(x_ref, c_ref, out_ref):
    ct = c_ref[...]
    nc = jnp.sum(ct * ct, axis=0, keepdims=True)
    nc_col = jnp.swapaxes(nc, 0, 1)
    ct2 = ct + ct
    for b in range(_B_BLK):
        xb = x_ref[b]
        nx = jnp.sum(xb * xb, axis=0)[None, :]
        dot2 = jax.lax.dot_general(
            ct2, xb, (((0,), (0,)), ((), ())),
            preferred_element_type=jnp.float32,
        )
        d = (nx + nc_col) - dot2
        idx = jnp.argmin(d, axis=0)
        out_ref[b, 0, :] = idx


def kernel(x, centroids):
    xt = jnp.swapaxes(x, 1, 2)
    ct = centroids.T
    out = pl.pallas_call(
        _assign_kernel,
        grid=(_B // _B_BLK,),
        in_specs=[
            pl.BlockSpec((_B_BLK, _K, _R), lambda i: (i, 0, 0)),
            pl.BlockSpec((_K, _C), lambda i: (0, 0)),
        ],
        out_specs=pl.BlockSpec((_B_BLK, 1, _R), lambda i: (i, 0, 0)),
        out_shape=jax.ShapeDtypeStruct((_B, 1, _R), jnp.int32),
    )(xt, ct)
    return out.reshape(_B, _R)


# speedup vs baseline: 2.0071x; 1.0882x over previous
import jax
import jax.numpy as jnp
from jax.experimental import pallas as pl
from jax.experimental.pallas import tpu as pltpu

_B = 16
_R = 576
_K = 64
_C = 1024


_B_BLK = 16


def _assign_kernel(x_ref, c_ref, out_ref):
    ct = c_ref[...]
    nc = jnp.sum(ct * ct, axis=0, keepdims=True)
    nc_col = jnp.swapaxes(nc, 0, 1)
    ct2 = ct + ct
    for p in range(_B_BLK // 2):
        xcat = jnp.concatenate([x_ref[2 * p], x_ref[2 * p + 1]], axis=1)
        nx = jnp.sum(xcat * xcat, axis=0)[None, :]
        dot2 = jax.lax.dot_general(
            ct2, xcat, (((0,), (0,)), ((), ())),
            preferred_element_type=jnp.float32,
        )
        d = (nx + nc_col) - dot2
        idx = jnp.argmin(d, axis=0)
        out_ref[2 * p, 0, :] = idx[:_R]
        out_ref[2 * p + 1, 0, :] = idx[_R:]


def kernel(x, centroids):
    xt = jnp.swapaxes(x, 1, 2)
    ct = centroids.T
    out = pl.pallas_call(
        _assign_kernel,
        grid=(_B // _B_BLK,),
        in_specs=[
            pl.BlockSpec((_B_BLK, _K, _R), lambda i: (i, 0, 0)),
            pl.BlockSpec((_K, _C), lambda i: (0, 0)),
        ],
        out_specs=pl.BlockSpec((_B_BLK, 1, _R), lambda i: (i, 0, 0)),
        out_shape=jax.ShapeDtypeStruct((_B, 1, _R), jnp.int32),
    )(xt, ct)
    return out.reshape(_B, _R)
